# Initial kernel scaffold; baseline (speedup 1.0000x reference)
import jax
import jax.numpy as jnp
from jax import lax
from jax.experimental import pallas as pl
from jax.experimental.pallas import tpu as pltpu
from jax.experimental.pallas import tpu_sc as plsc

N = 10000
E = 320000
D = 128

NC = 2
NS = 16
NW = NC * NS
EPW = E // NW
CH = 128
NFULL = EPW // CH
REM = EPW - NFULL * CH
ROWS_PT = N // NS
ZR = 125

_MB = 1000


def _agg_body(src_hbm, dst_hbm, m_hbm, out_hbm,
              srcv, dstv, rows, srcr, dstr, rowsr, zbuf, acc, sem):
    c = lax.axis_index("c")
    s = lax.axis_index("s")

    zero16 = jnp.zeros((16,), jnp.float32)

    def _zfill(i, carry):
        for j in range(8):
            zbuf[i, pl.ds(j * 16, 16)] = zero16
        return carry

    lax.fori_loop(0, ZR, _zfill, 0)
    for k in range(ROWS_PT // ZR):
        pltpu.sync_copy(zbuf, acc.at[pl.ds(s * ROWS_PT + k * ZR, ZR), :])
    plsc.subcore_barrier()

    ebase = c * (E // NC) + s * EPW

    def _chunk(i, carry):
        base = ebase + i * CH
        pltpu.sync_copy(src_hbm.at[pl.ds(base, CH)], srcv)
        pltpu.sync_copy(dst_hbm.at[pl.ds(base, CH)], dstv)
        pltpu.async_copy(m_hbm.at[srcv], rows, sem).wait()
        pltpu.sync_copy(rows, acc.at[dstv], add=True)
        return carry

    lax.fori_loop(0, NFULL, _chunk, 0)

    base = ebase + NFULL * CH
    pltpu.sync_copy(src_hbm.at[pl.ds(base, REM)], srcr)
    pltpu.sync_copy(dst_hbm.at[pl.ds(base, REM)], dstr)
    pltpu.async_copy(m_hbm.at[srcr], rowsr, sem).wait()
    pltpu.sync_copy(rowsr, acc.at[dstr], add=True)

    plsc.subcore_barrier()

    pltpu.sync_copy(acc.at[pl.ds(s * ROWS_PT, ROWS_PT), :],
                    out_hbm.at[c, pl.ds(s * ROWS_PT, ROWS_PT), :])


@jax.jit
def _agg(src, dst, m):
    mesh = plsc.VectorSubcoreMesh(core_axis_name="c", subcore_axis_name="s")
    return pl.kernel(
        _agg_body,
        out_type=jax.ShapeDtypeStruct((NC, N, D), jnp.float32),
        mesh=mesh,
        scratch_types=[
            pltpu.VMEM((CH,), jnp.int32),
            pltpu.VMEM((CH,), jnp.int32),
            pltpu.VMEM((CH, D), jnp.float32),
            pltpu.VMEM((REM,), jnp.int32),
            pltpu.VMEM((REM,), jnp.int32),
            pltpu.VMEM((REM, D), jnp.float32),
            pltpu.VMEM((ZR, D), jnp.float32),
            pltpu.VMEM_SHARED((N, D), jnp.float32),
            pltpu.SemaphoreType.DMA,
        ],
    )(src, dst, m)


def _mm_body(x_ref, w_ref, o_ref):
    o_ref[...] = jnp.dot(x_ref[...], w_ref[...],
                         preferred_element_type=jnp.float32)


@jax.jit
def _mm(x, w):
    return pl.pallas_call(
        _mm_body,
        grid=(N // _MB,),
        in_specs=[
            pl.BlockSpec((_MB, D), lambda i: (i, 0)),
            pl.BlockSpec((D, D), lambda i: (0, 0)),
        ],
        out_specs=pl.BlockSpec((_MB, D), lambda i: (i, 0)),
        out_shape=jax.ShapeDtypeStruct((N, D), jnp.float32),
    )(x, w)


def _combine_mm_body(p_ref, b_ref, w_ref, o_ref):
    x = jnp.maximum(p_ref[0] + p_ref[1] + b_ref[...], 0.0)
    o_ref[...] = jnp.dot(x, w_ref[...], preferred_element_type=jnp.float32)


@jax.jit
def _combine_mm(p, b, w):
    return pl.pallas_call(
        _combine_mm_body,
        grid=(N // _MB,),
        in_specs=[
            pl.BlockSpec((NC, _MB, D), lambda i: (0, i, 0)),
            pl.BlockSpec((1, D), lambda i: (0, 0)),
            pl.BlockSpec((D, D), lambda i: (0, 0)),
        ],
        out_specs=pl.BlockSpec((_MB, D), lambda i: (i, 0)),
        out_shape=jax.ShapeDtypeStruct((N, D), jnp.float32),
    )(p, b, w)


def _head_body(p_ref, b2_ref, wh1_ref, bh1_ref, wh2_ref, bh2_ref,
               wc_ref, bc_ref, o_ref):
    x = jnp.maximum(p_ref[0] + p_ref[1] + b2_ref[...], 0.0)
    h = jnp.maximum(
        jnp.dot(x, wh1_ref[...], preferred_element_type=jnp.float32)
        + bh1_ref[...], 0.0)
    h = jnp.maximum(
        jnp.dot(h, wh2_ref[...], preferred_element_type=jnp.float32)
        + bh2_ref[...], 0.0)
    av = (jnp.dot(h, wc_ref[...], preferred_element_type=jnp.float32)
          + bc_ref[...])
    col = lax.broadcasted_iota(jnp.int32, av.shape, 1)
    adv_sum = jnp.sum(jnp.where(col < 5, av, 0.0), axis=1, keepdims=True)
    val = jnp.sum(jnp.where(col == 5, av, 0.0), axis=1, keepdims=True)
    o_ref[...] = val + av - adv_sum * (1.0 / 5.0)


@jax.jit
def _head(p, b2, wh1, bh1, wh2, bh2, wc, bc):
    return pl.pallas_call(
        _head_body,
        grid=(N // _MB,),
        in_specs=[
            pl.BlockSpec((NC, _MB, D), lambda i: (0, i, 0)),
            pl.BlockSpec((1, D), lambda i: (0, 0)),
            pl.BlockSpec((D, D), lambda i: (0, 0)),
            pl.BlockSpec((1, D), lambda i: (0, 0)),
            pl.BlockSpec((D, D), lambda i: (0, 0)),
            pl.BlockSpec((1, D), lambda i: (0, 0)),
            pl.BlockSpec((D, 8), lambda i: (0, 0)),
            pl.BlockSpec((1, 8), lambda i: (0, 0)),
        ],
        out_specs=pl.BlockSpec((_MB, 8), lambda i: (i, 0)),
        out_shape=jax.ShapeDtypeStruct((N, 8), jnp.float32),
    )(p, b2, wh1, bh1, wh2, bh2, wc, bc)


def kernel(edge_index, entity_embeddings, W1, b1, W2, b2,
           Wh1, bh1, Wh2, bh2, Wadv, badv, Wval, bval):
    src = edge_index[0]
    dst = edge_index[1]
    wc = jnp.concatenate([Wadv, Wval, jnp.zeros((D, 2), jnp.float32)], axis=1)
    bc = jnp.concatenate([badv, bval, jnp.zeros((2,), jnp.float32)])[None, :]

    m1 = _mm(entity_embeddings, W1)
    p1 = _agg(src, dst, m1)
    m2 = _combine_mm(p1, b1[None, :], W2)
    p2 = _agg(src, dst, m2)
    q8 = _head(p2, b2[None, :], Wh1, bh1[None, :], Wh2, bh2[None, :], wc, bc)
    return q8[:, :5]


# speedup vs baseline: 5.7244x; 5.7244x over previous
import jax
import jax.numpy as jnp
from jax import lax
from jax.experimental import pallas as pl
from jax.experimental.pallas import tpu as pltpu
from jax.experimental.pallas import tpu_sc as plsc

N = 10000
E = 320000
D = 128

NC = 2
NS = 16
NW = NC * NS
EPW = E // NW
CH = 128
NFULL = EPW // CH
REM = EPW - NFULL * CH
ST = 640
ST_LAST = N - 15 * ST

_MB = 1000


def _agg_body(src_hbm, dst_hbm, m_hbm, out_hbm,
              srcv, dstv, rows, srcr, dstr, rowsr, acc, sem):
    c = lax.axis_index("c")
    s = lax.axis_index("s")

    zero16 = jnp.zeros((16,), jnp.float32)

    def _zfill(i, carry):
        for j in range(8):
            rows[i, pl.ds(j * 16, 16)] = zero16
        return carry

    lax.fori_loop(0, CH, _zfill, 0)
    def _zfill_r(i, carry):
        for j in range(8):
            rowsr[i, pl.ds(j * 16, 16)] = zero16
        return carry

    lax.fori_loop(0, REM, _zfill_r, 0)
    ofs = pl.multiple_of(s * ST, 8)

    @pl.when(s < 15)
    def _():
        for k in range(ST // CH):
            pltpu.sync_copy(rows, acc.at[pl.ds(ofs + k * CH, CH), :])

    @pl.when(s == 15)
    def _():
        for k in range(3):
            pltpu.sync_copy(rows, acc.at[pl.ds(15 * ST + k * CH, CH), :])
        pltpu.sync_copy(rowsr, acc.at[pl.ds(15 * ST + 3 * CH, REM), :])

    plsc.subcore_barrier()

    ebase = c * (E // NC) + s * EPW

    def _chunk(i, carry):
        base = pl.multiple_of(ebase + i * CH, 8)
        pltpu.sync_copy(src_hbm.at[pl.ds(base, CH)], srcv)
        pltpu.sync_copy(dst_hbm.at[pl.ds(base, CH)], dstv)
        pltpu.async_copy(m_hbm.at[srcv], rows, sem).wait()
        pltpu.sync_copy(rows, acc.at[dstv], add=True)
        return carry

    lax.fori_loop(0, NFULL, _chunk, 0)

    base = pl.multiple_of(ebase + NFULL * CH, 8)
    pltpu.sync_copy(src_hbm.at[pl.ds(base, REM)], srcr)
    pltpu.sync_copy(dst_hbm.at[pl.ds(base, REM)], dstr)
    pltpu.async_copy(m_hbm.at[srcr], rowsr, sem).wait()
    pltpu.sync_copy(rowsr, acc.at[dstr], add=True)

    plsc.subcore_barrier()

    @pl.when(s < 15)
    def _():
        pltpu.sync_copy(acc.at[pl.ds(ofs, ST), :],
                        out_hbm.at[c, pl.ds(ofs, ST), :])

    @pl.when(s == 15)
    def _():
        pltpu.sync_copy(acc.at[pl.ds(15 * ST, ST_LAST), :],
                        out_hbm.at[c, pl.ds(15 * ST, ST_LAST), :])


@jax.jit
def _agg(src, dst, m):
    mesh = plsc.VectorSubcoreMesh(core_axis_name="c", subcore_axis_name="s")
    return pl.kernel(
        _agg_body,
        out_type=jax.ShapeDtypeStruct((NC, N, D), jnp.float32),
        mesh=mesh,
        scratch_types=[
            pltpu.VMEM((CH,), jnp.int32),
            pltpu.VMEM((CH,), jnp.int32),
            pltpu.VMEM((CH, D), jnp.float32),
            pltpu.VMEM((REM,), jnp.int32),
            pltpu.VMEM((REM,), jnp.int32),
            pltpu.VMEM((REM, D), jnp.float32),
            pltpu.VMEM_SHARED((N, D), jnp.float32),
            pltpu.SemaphoreType.DMA,
        ],
    )(src, dst, m)


def _mm_body(x_ref, w_ref, o_ref):
    o_ref[...] = jnp.dot(x_ref[...], w_ref[...],
                         preferred_element_type=jnp.float32)


@jax.jit
def _mm(x, w):
    return pl.pallas_call(
        _mm_body,
        grid=(N // _MB,),
        in_specs=[
            pl.BlockSpec((_MB, D), lambda i: (i, 0)),
            pl.BlockSpec((D, D), lambda i: (0, 0)),
        ],
        out_specs=pl.BlockSpec((_MB, D), lambda i: (i, 0)),
        out_shape=jax.ShapeDtypeStruct((N, D), jnp.float32),
    )(x, w)


def _combine_mm_body(p_ref, b_ref, w_ref, o_ref):
    x = jnp.maximum(p_ref[0] + p_ref[1] + b_ref[...], 0.0)
    o_ref[...] = jnp.dot(x, w_ref[...], preferred_element_type=jnp.float32)


@jax.jit
def _combine_mm(p, b, w):
    return pl.pallas_call(
        _combine_mm_body,
        grid=(N // _MB,),
        in_specs=[
            pl.BlockSpec((NC, _MB, D), lambda i: (0, i, 0)),
            pl.BlockSpec((1, D), lambda i: (0, 0)),
            pl.BlockSpec((D, D), lambda i: (0, 0)),
        ],
        out_specs=pl.BlockSpec((_MB, D), lambda i: (i, 0)),
        out_shape=jax.ShapeDtypeStruct((N, D), jnp.float32),
    )(p, b, w)


def _head_body(p_ref, b2_ref, wh1_ref, bh1_ref, wh2_ref, bh2_ref,
               wc_ref, bc_ref, o_ref):
    x = jnp.maximum(p_ref[0] + p_ref[1] + b2_ref[...], 0.0)
    h = jnp.maximum(
        jnp.dot(x, wh1_ref[...], preferred_element_type=jnp.float32)
        + bh1_ref[...], 0.0)
    h = jnp.maximum(
        jnp.dot(h, wh2_ref[...], preferred_element_type=jnp.float32)
        + bh2_ref[...], 0.0)
    av = (jnp.dot(h, wc_ref[...], preferred_element_type=jnp.float32)
          + bc_ref[...])
    col = lax.broadcasted_iota(jnp.int32, av.shape, 1)
    adv_sum = jnp.sum(jnp.where(col < 5, av, 0.0), axis=1, keepdims=True)
    val = jnp.sum(jnp.where(col == 5, av, 0.0), axis=1, keepdims=True)
    o_ref[...] = val + av - adv_sum * (1.0 / 5.0)


@jax.jit
def _head(p, b2, wh1, bh1, wh2, bh2, wc, bc):
    return pl.pallas_call(
        _head_body,
        grid=(N // _MB,),
        in_specs=[
            pl.BlockSpec((NC, _MB, D), lambda i: (0, i, 0)),
            pl.BlockSpec((1, D), lambda i: (0, 0)),
            pl.BlockSpec((D, D), lambda i: (0, 0)),
            pl.BlockSpec((1, D), lambda i: (0, 0)),
            pl.BlockSpec((D, D), lambda i: (0, 0)),
            pl.BlockSpec((1, D), lambda i: (0, 0)),
            pl.BlockSpec((D, 8), lambda i: (0, 0)),
            pl.BlockSpec((1, 8), lambda i: (0, 0)),
        ],
        out_specs=pl.BlockSpec((_MB, 8), lambda i: (i, 0)),
        out_shape=jax.ShapeDtypeStruct((N, 8), jnp.float32),
    )(p, b2, wh1, bh1, wh2, bh2, wc, bc)


def kernel(edge_index, entity_embeddings, W1, b1, W2, b2,
           Wh1, bh1, Wh2, bh2, Wadv, badv, Wval, bval):
    src = edge_index[0]
    dst = edge_index[1]
    wc = jnp.concatenate([Wadv, Wval, jnp.zeros((D, 2), jnp.float32)], axis=1)
    bc = jnp.concatenate([badv, bval, jnp.zeros((2,), jnp.float32)])[None, :]

    m1 = _mm(entity_embeddings, W1)
    p1 = _agg(src, dst, m1)
    m2 = _combine_mm(p1, b1[None, :], W2)
    p2 = _agg(src, dst, m2)
    q8 = _head(p2, b2[None, :], Wh1, bh1[None, :], Wh2, bh2[None, :], wc, bc)
    return q8[:, :5]
